# baseline (speedup 1.0000x reference)
import jax
import jax.numpy as jnp
from jax import lax
from jax.experimental import pallas as pl
from jax.experimental.pallas import tpu as pltpu
from jax.experimental.pallas import tpu_sc as plsc

_EMB = 16
_HID = 128
_IN = 3217
_NWORDS = 200
_ROWS_PER_TILE = 8
_SLAB = _ROWS_PER_TILE * _IN
_AUX = 544

_mesh = plsc.VectorSubcoreMesh(
    core_axis_name="c", subcore_axis_name="s", num_cores=1
)

_SC_CFG = dict(
    out_type=jax.ShapeDtypeStruct((16,), jnp.float32),
    mesh=_mesh,
    compiler_params=pltpu.CompilerParams(
        needs_layout_passes=False, use_tc_tiling_on_sc=False
    ),
    scratch_types=[
        pltpu.VMEM((208,), jnp.int32),
        pltpu.VMEM((_NWORDS * 16,), jnp.int32),
        pltpu.VMEM((16,), jnp.int32),
        pltpu.VMEM((208, 128), jnp.float32),
        pltpu.VMEM((16,), jnp.float32),
        pltpu.VMEM((_SLAB,), jnp.float32),
        pltpu.VMEM((_AUX,), jnp.float32),
        pltpu.VMEM((16, 16), jnp.float32),
        pltpu.VMEM((16,), jnp.float32),
        pltpu.VMEM_SHARED((16, 16), jnp.float32),
        pltpu.SemaphoreType.DMA,
        pltpu.SemaphoreType.DMA,
        pltpu.SemaphoreType.DMA,
        pltpu.SemaphoreType.DMA,
        pltpu.SemaphoreType.DMA,
    ],
)


def _sc_body(word_ref, spk_ref, w1_ref, ridx_ref, colidx_ref, sidx_ref,
             aux_ref, out_ref, ridx_v, colidx_v, sidx_v, rows_v, spk_v, w1_v,
             aux_v, psum_v, res_v, part_sh, sem0, sem1, sem2, sem3, sem4):
    s = lax.axis_index("s")

    pltpu.sync_copy(ridx_ref, ridx_v)
    pltpu.sync_copy(colidx_ref, colidx_v)
    pltpu.sync_copy(sidx_ref, sidx_v)
    cpa = pltpu.async_copy(aux_ref, aux_v, sem0)
    cp0 = pltpu.async_copy(
        word_ref.at[ridx_v.at[pl.ds(0, 128)]], rows_v.at[pl.ds(0, 128)], sem1
    )
    cp1 = pltpu.async_copy(
        word_ref.at[ridx_v.at[pl.ds(128, 80)]], rows_v.at[pl.ds(128, 80)], sem2
    )
    cp2 = pltpu.async_copy(spk_ref.at[sidx_v], spk_v, sem3)
    cpw = pltpu.async_copy(w1_ref.at[pl.ds(s * _SLAB, _SLAB)], w1_v, sem4)
    cpa.wait()
    cp0.wait()
    cp1.wait()
    cp2.wait()
    cpw.wait()

    x0 = spk_v[...]
    accs = tuple(
        w1_v[pl.ds(r * _IN, 16)] * x0 for r in range(_ROWS_PER_TILE)
    )

    def dot_body(j, accs):
        colvec = colidx_v[pl.ds((j - 1) * 16, 16)]
        rowvec = jnp.full((16,), j - 1, jnp.int32)
        xj = plsc.load_gather(rows_v, [rowvec, colvec])
        return tuple(
            accs[r] + w1_v[pl.ds(r * _IN + j * 16, 16)] * xj
            for r in range(_ROWS_PER_TILE)
        )

    accs = lax.fori_loop(1, _NWORDS + 1, dot_body, accs)

    qb = aux_v[pl.ds(512, 16)]
    quant = qb[0]
    bvec = aux_v[pl.ds(s * 16, 16)]
    w2vec = aux_v[pl.ds(256 + s * 16, 16)]
    partial = jnp.float32(0.0)
    for r in range(_ROWS_PER_TILE):
        wlast = w1_v[pl.ds(r * _IN + _IN - 16, 16)]
        h = jnp.sum(accs[r]) + quant * wlast[15] + bvec[r]
        h = jnp.maximum(h, 0.0)
        partial = partial + h * w2vec[r]

    res_v[...] = jnp.full((16,), partial, jnp.float32)
    pltpu.sync_copy(res_v, part_sh.at[s])
    plsc.subcore_barrier()

    @pl.when(s == 0)
    def _():
        pltpu.sync_copy(part_sh, psum_v)
        tot = psum_v[0]
        for i in range(1, 16):
            tot = tot + psum_v[i]
        z = tot + aux_v[pl.ds(512, 16)][1]
        res_v[...] = 1.0 / (1.0 + jnp.exp(-z))
        pltpu.sync_copy(res_v, out_ref)


_sc_forward = pl.kernel(_sc_body, **_SC_CFG)


def kernel(context_indices, speaker, col_three_indices, quant, sentiment,
           word_emb, speaker_emb, W1, b1, W2, b2):
    del sentiment
    ctx = context_indices.astype(jnp.int32)
    c3 = col_three_indices.astype(jnp.int32)
    words = jnp.concatenate([ctx, c3])
    words208 = jnp.concatenate([words, jnp.broadcast_to(c3, (8,))])
    ridx = words208 // 8
    lane = jnp.arange(16, dtype=jnp.int32)
    colidx = ((words % 8) * 16)[:, None] + lane[None, :]
    sidx = speaker.astype(jnp.int32) * 16 + lane
    b1p = jnp.pad(b1.reshape(16, 8), ((0, 0), (0, 8)))
    w2p = jnp.pad(W2.reshape(16, 8), ((0, 0), (0, 8)))
    aux = jnp.concatenate([
        b1p.reshape(-1), w2p.reshape(-1),
        quant.astype(jnp.float32), b2.astype(jnp.float32),
        jnp.zeros((_AUX - 514,), jnp.float32),
    ])
    out16 = _sc_forward(
        word_emb.reshape(125000, 128),
        speaker_emb.reshape(-1),
        W1.reshape(-1),
        ridx,
        colidx.reshape(-1),
        sidx,
        aux,
    )
    return out16[:1].reshape(1, 1)


# speedup vs baseline: 21.0697x; 21.0697x over previous
import functools

import jax
import jax.numpy as jnp
from jax import lax
from jax.experimental import pallas as pl
from jax.experimental.pallas import tpu as pltpu

_K = 8
_STEPS = 25
_HID = 128


def _tc_body(pidx, pq, *refs):
    wblks = refs[0:8]
    spk_blk = refs[8]
    w1segs = refs[9:17]
    w1_spk = refs[17]
    w1_qnt = refs[18]
    b1_ref = refs[19]
    w2_ref = refs[20]
    out_ref = refs[21]
    h_ref = refs[22]
    i = pl.program_id(0)

    lane = lax.broadcasted_iota(jnp.int32, (1, 128), 1)

    def col_contrib(blk, col, w1seg):
        oh = (lane == col).astype(jnp.float32)
        emb = lax.dot_general(blk[...], oh, (((1,), (1,)), ((), ())),
                              preferred_element_type=jnp.float32)
        return lax.dot_general(emb, w1seg[...], (((0,), (0,)), ((), ())),
                               preferred_element_type=jnp.float32)

    @pl.when(i == 0)
    def _():
        h_ref[...] = col_contrib(spk_blk, pidx[0] % 128, w1_spk)

    acc = h_ref[...]
    for k in range(_K):
        w = pidx[1 + i * _K + k]
        acc = acc + col_contrib(wblks[k], w % 128, w1segs[k])
    h_ref[...] = acc

    @pl.when(i == _STEPS - 1)
    def _():
        h = h_ref[...] + pq[0] * w1_qnt[0:1, :] + b1_ref[...]
        h = jnp.maximum(h, 0.0)
        s = jnp.sum(h * w2_ref[...])
        out_ref[...] = jnp.full((1, 1), 1.0 / (1.0 + jnp.exp(-(s + pq[1]))))


def kernel(context_indices, speaker, col_three_indices, quant, sentiment,
           word_emb, speaker_emb, W1, b1, W2, b2):
    del sentiment
    ctx = context_indices.astype(jnp.int32)
    c3 = col_three_indices.astype(jnp.int32)
    pidx = jnp.concatenate([speaker.astype(jnp.int32), ctx, c3])
    pq = jnp.concatenate([quant.astype(jnp.float32), b2.astype(jnp.float32)])

    wordT = word_emb.T
    spkT = speaker_emb.T
    w1T = W1.T
    b1r = b1.reshape(1, _HID)

    word_specs = [
        pl.BlockSpec(
            (16, 128),
            functools.partial(
                lambda kk, i, pidx, pq: (0, pidx[1 + i * _K + kk] // 128), k
            ),
        )
        for k in range(_K)
    ]
    spk_spec = pl.BlockSpec((16, 128), lambda i, pidx, pq: (0, pidx[0] // 128))
    w1seg_specs = [
        pl.BlockSpec(
            (16, 128),
            functools.partial(lambda kk, i, pidx, pq: (i * _K + kk + 1, 0), k),
        )
        for k in range(_K)
    ]
    w1spk_spec = pl.BlockSpec((16, 128), lambda i, pidx, pq: (0, 0))
    w1qnt_spec = pl.BlockSpec((16, 128), lambda i, pidx, pq: (201, 0))
    b1_spec = pl.BlockSpec((1, _HID), lambda i, pidx, pq: (0, 0))
    w2_spec = pl.BlockSpec((1, _HID), lambda i, pidx, pq: (0, 0))
    out_spec = pl.BlockSpec((1, 1), lambda i, pidx, pq: (0, 0))

    grid_spec = pltpu.PrefetchScalarGridSpec(
        num_scalar_prefetch=2,
        grid=(_STEPS,),
        in_specs=word_specs + [spk_spec] + w1seg_specs
        + [w1spk_spec, w1qnt_spec, b1_spec, w2_spec],
        out_specs=out_spec,
        scratch_shapes=[pltpu.VMEM((1, _HID), jnp.float32)],
    )
    out = pl.pallas_call(
        _tc_body,
        grid_spec=grid_spec,
        out_shape=jax.ShapeDtypeStruct((1, 1), jnp.float32),
    )(
        pidx, pq,
        *([wordT] * _K), spkT, *([w1T] * _K), w1T, w1T, b1r, W2,
    )
    return out
